# baseline (speedup 1.0000x reference)
import functools

import jax
import jax.numpy as jnp
from jax import lax
from jax.experimental import pallas as pl
from jax.experimental.pallas import tpu as pltpu
from jax.experimental.pallas import tpu_sc as plsc

_NTEC = 16
_EK = 80


def _sc_mesh():
    return plsc.VectorSubcoreMesh(core_axis_name="c", subcore_axis_name="s")


_SC_PARAMS = pltpu.CompilerParams(needs_layout_passes=False)


def _make_deg_kernel(E, NPAD):
    EPT = E // _NTEC
    RPT = NPAD // _NTEC

    @functools.partial(
        pl.kernel,
        out_type=jax.ShapeDtypeStruct((2 * NPAD,), jnp.float32),
        mesh=_sc_mesh(),
        compiler_params=_SC_PARAMS,
        scratch_types=[
            pltpu.VMEM((NPAD,), jnp.float32),
            pltpu.VMEM_SHARED((_NTEC, NPAD), jnp.float32),
            pltpu.VMEM((EPT,), jnp.int32),
            pltpu.VMEM((_NTEC, RPT), jnp.float32),
            pltpu.VMEM((RPT,), jnp.float32),
        ],
    )
    def deg_kernel(ei_hbm, o_hbm, accv, stage_sh, idx_all, pbuf, obuf):
        cid = lax.axis_index("c")
        sid = lax.axis_index("s")
        zeros16 = jnp.zeros((16,), jnp.float32)
        ones16 = jnp.full((16,), 1.0, jnp.float32)

        def _z(i, c):
            accv[pl.ds(i * 16, 16)] = zeros16
            return c
        lax.fori_loop(0, NPAD // 16, _z, 0)

        pltpu.sync_copy(ei_hbm.at[pl.ds(cid * E + sid * EPT, EPT)], idx_all)

        def _e(i, c):
            v = idx_all[pl.ds(i * 16, 16)]
            plsc.addupdate_scatter(accv, [v], ones16)
            return c
        lax.fori_loop(0, EPT // 16, _e, 0)

        pltpu.sync_copy(accv, stage_sh.at[sid])
        plsc.subcore_barrier()
        pltpu.sync_copy(stage_sh.at[:, pl.ds(sid * RPT, RPT)], pbuf)

        def _s(i, c):
            a = pbuf[0, pl.ds(i * 16, 16)]
            for j in range(1, _NTEC):
                a = a + pbuf[j, pl.ds(i * 16, 16)]
            obuf[pl.ds(i * 16, 16)] = a
            return c
        lax.fori_loop(0, RPT // 16, _s, 0)

        pltpu.sync_copy(obuf, o_hbm.at[pl.ds(cid * NPAD + sid * RPT, RPT)])

    return deg_kernel


def _make_msg_kernel(NP, E, DH):
    EPT = E // _NTEC
    NCH = EPT // _EK
    assert NCH % 2 == 0, "pipeline assumes an even chunk count"
    RPT = NP // _NTEC
    ZR = 32
    NB = (NCH - 2) // 2

    @functools.partial(
        pl.kernel,
        out_type=jax.ShapeDtypeStruct((2 * NP, DH), jnp.float32),
        mesh=_sc_mesh(),
        compiler_params=_SC_PARAMS,
        scratch_types=[
            pltpu.VMEM_SHARED((NP, DH), jnp.float32),
            pltpu.VMEM((_EK, DH), jnp.float32),
            pltpu.VMEM((_EK, DH), jnp.float32),
            pltpu.VMEM((EPT,), jnp.int32),
            pltpu.VMEM((_EK,), jnp.int32),
            pltpu.VMEM((_EK,), jnp.int32),
            pltpu.VMEM((ZR, DH), jnp.float32),
            pltpu.SemaphoreType.DMA,
            pltpu.SemaphoreType.DMA,
            pltpu.SemaphoreType.DMA,
            pltpu.SemaphoreType.DMA,
            pltpu.SemaphoreType.DMA,
            pltpu.SemaphoreType.DMA,
            pltpu.SemaphoreType.DMA,
            pltpu.SemaphoreType.DMA,
        ],
    )
    def msg_kernel(g_hbm, ei_hbm, o_hbm,
                   acc_sh, rowsA, rowsB, gidx, dstA, dstB, zbuf,
                   sI, sZ, sgA, sgB, ssA, ssB, sdA, sdB):
        cid = lax.axis_index("c")
        sid = lax.axis_index("s")
        zeros16 = jnp.zeros((16,), jnp.float32)
        base = sid * EPT

        idx_cp = pltpu.async_copy(ei_hbm.at[pl.ds(base, EPT)], gidx, sI)

        def _z(i, c):
            for m in range(DH // 16):
                zbuf[i, pl.ds(m * 16, 16)] = zeros16
            return c
        lax.fori_loop(0, ZR, _z, 0)
        for k in range(RPT // ZR):
            pltpu.async_copy(zbuf, acc_sh.at[pl.ds(sid * RPT + k * ZR, ZR)], sZ)
        idx_cp.wait()

        cbase = cid * 8

        def _gi(i, c):
            v = gidx[pl.ds(i * 16, 16)]
            gidx[pl.ds(i * 16, 16)] = ((v >> 3) << 4) + (v & 7) + cbase
            return c
        lax.fori_loop(0, EPT // 16, _gi, 0)
        for k in range(RPT // ZR):
            pltpu.make_async_copy(
                zbuf, acc_sh.at[pl.ds(sid * RPT, ZR)], sZ).wait()
        plsc.subcore_barrier()

        def _gather(j, rows, sem):
            pltpu.async_copy(g_hbm.at[gidx.at[pl.ds(j * _EK, _EK)]], rows, sem)

        def _wait_gather(rows, sem):
            pltpu.make_async_copy(
                g_hbm.at[gidx.at[pl.ds(0, _EK)]], rows, sem).wait()

        def _scatter(rows, db, sem):
            pltpu.async_copy(rows, acc_sh.at[db], sem, add=True)

        def _wait_scatter(rows, db, sem):
            pltpu.make_async_copy(rows, acc_sh.at[db], sem).wait()

        def _dstload(j, db, sem):
            pltpu.async_copy(ei_hbm.at[pl.ds(E + base + j * _EK, _EK)], db, sem)

        def _wait_dstload(db, sem):
            pltpu.make_async_copy(
                ei_hbm.at[pl.ds(E, _EK)], db, sem).wait()

        pltpu.sync_copy(ei_hbm.at[pl.ds(E + base, _EK)], dstA)
        _gather(0, rowsA, sgA)

        def _pair(i, c):
            j0 = 2 * i

            @pl.when(i > 0)
            def _():
                _wait_scatter(rowsB, dstB, ssB)
            _dstload(j0 + 1, dstB, sdB)
            _gather(j0 + 1, rowsB, sgB)

            _wait_gather(rowsA, sgA)

            @pl.when(i > 0)
            def _():
                _wait_dstload(dstA, sdA)
            _scatter(rowsA, dstA, ssA)

            _wait_gather(rowsB, sgB)
            _wait_scatter(rowsA, dstA, ssA)
            _dstload(j0 + 2, dstA, sdA)
            _gather(j0 + 2, rowsA, sgA)

            _wait_dstload(dstB, sdB)
            _scatter(rowsB, dstB, ssB)
            return c
        lax.fori_loop(0, NB, _pair, 0)

        _wait_scatter(rowsB, dstB, ssB)
        _dstload(NCH - 1, dstB, sdB)
        _gather(NCH - 1, rowsB, sgB)
        _wait_gather(rowsA, sgA)
        _wait_dstload(dstA, sdA)
        _scatter(rowsA, dstA, ssA)
        _wait_gather(rowsB, sgB)
        _wait_dstload(dstB, sdB)
        _scatter(rowsB, dstB, ssB)
        _wait_scatter(rowsA, dstA, ssA)
        _wait_scatter(rowsB, dstB, ssB)

        plsc.subcore_barrier()
        r0 = sid * RPT
        pltpu.sync_copy(acc_sh.at[pl.ds(r0, RPT)],
                        o_hbm.at[pl.ds(cid * NP + r0, RPT)])

    return msg_kernel


def _half_shuffle(gb, B, DH):
    return gb.reshape(B // 8, 8, 2, DH).swapaxes(1, 2).reshape(2 * B, DH)


def _tc1_body(x_ref, w_ref, deg_ref, o_ref):
    s = lax.rsqrt(jnp.maximum(deg_ref[...], 1.0))
    g = jnp.dot(x_ref[...] * s, w_ref[...],
                preferred_element_type=jnp.float32)
    o_ref[...] = _half_shuffle(g, g.shape[0], g.shape[1] // 2)


def _tc1(x, W1, deg_out, NP, B):
    N, D = x.shape
    nb = NP // B
    return pl.pallas_call(
        _tc1_body,
        grid=(nb,),
        in_specs=[pl.BlockSpec((B, D), lambda i: (i, 0)),
                  pl.BlockSpec((D, D), lambda i: (0, 0)),
                  pl.BlockSpec((B, 1), lambda i: (i, 0))],
        out_specs=pl.BlockSpec((2 * B, D // 2), lambda i: (i, 0)),
        out_shape=jax.ShapeDtypeStruct((2 * NP, D // 2), jnp.float32),
    )(x, W1, deg_out)


def _tc2_body(a0_ref, a1_ref, di_ref, do_ref, b_ref, w_ref, o_ref):
    si = lax.rsqrt(jnp.maximum(di_ref[...], 1.0))
    so = lax.rsqrt(jnp.maximum(do_ref[...], 1.0))
    b = b_ref[...]
    h0 = jax.nn.relu(a0_ref[...] * si + b[:, :128]) * so
    h1 = jax.nn.relu(a1_ref[...] * si + b[:, 128:]) * so
    w = w_ref[...]
    g = (jnp.dot(h0, w[:128, :], preferred_element_type=jnp.float32)
         + jnp.dot(h1, w[128:, :], preferred_element_type=jnp.float32))
    o_ref[...] = _half_shuffle(g, g.shape[0], g.shape[1] // 2)


def _tc2(a_flat, deg_in, deg_out, b1, W2, NP, B):
    D = W2.shape[0]
    nb = NP // B
    return pl.pallas_call(
        _tc2_body,
        grid=(nb,),
        in_specs=[pl.BlockSpec((B, 128), lambda i: (i, 0)),
                  pl.BlockSpec((B, 128), lambda i, _nb=nb: (_nb + i, 0)),
                  pl.BlockSpec((B, 1), lambda i: (i, 0)),
                  pl.BlockSpec((B, 1), lambda i: (i, 0)),
                  pl.BlockSpec((1, D), lambda i: (0, 0)),
                  pl.BlockSpec((D, D), lambda i: (0, 0))],
        out_specs=pl.BlockSpec((2 * B, D // 2), lambda i: (i, 0)),
        out_shape=jax.ShapeDtypeStruct((2 * NP, D // 2), jnp.float32),
    )(a_flat, a_flat, deg_in, deg_out, b1, W2)


def _tc3_body(a0_ref, a1_ref, di_ref, b2_ref, wd1_ref, bd1_ref,
              wd2_ref, bd2_ref, wo_ref, bo_ref, o_ref, *, n_real):
    npad = a0_ref.shape[0]
    si = lax.rsqrt(jnp.maximum(di_ref[...], 1.0))
    mask = (lax.broadcasted_iota(jnp.int32, (npad, 1), 0)
            < n_real).astype(jnp.float32)
    b2 = b2_ref[...]
    h0 = jax.nn.relu(a0_ref[...] * si + b2[:, :128]) * mask
    h1 = jax.nn.relu(a1_ref[...] * si + b2[:, 128:]) * mask
    m0 = jnp.sum(h0, axis=0, keepdims=True) * (1.0 / n_real)
    m1 = jnp.sum(h1, axis=0, keepdims=True) * (1.0 / n_real)
    hg = jnp.concatenate([m0, m1], axis=1)
    z = jax.nn.relu(jnp.dot(hg, wd1_ref[...],
                            preferred_element_type=jnp.float32) + bd1_ref[...])
    z = jax.nn.relu(jnp.dot(z, wd2_ref[...],
                            preferred_element_type=jnp.float32) + bd2_ref[...])
    o_ref[...] = jnp.dot(z, wo_ref[...],
                         preferred_element_type=jnp.float32) + bo_ref[...]


def _tc3(c_flat, deg_in, b2, Wd1, bd1, Wd2, bd2, Wo, bo, N, NP):
    NC = Wo.shape[1]
    H3 = Wd1.shape[1]
    H4 = Wd2.shape[1]
    D = Wd1.shape[0]
    return pl.pallas_call(
        functools.partial(_tc3_body, n_real=N),
        grid=(1,),
        in_specs=[pl.BlockSpec((NP, 128), lambda i: (0, 0)),
                  pl.BlockSpec((NP, 128), lambda i: (1, 0)),
                  pl.BlockSpec((NP, 1), lambda i: (0, 0)),
                  pl.BlockSpec((1, D), lambda i: (0, 0)),
                  pl.BlockSpec((D, H3), lambda i: (0, 0)),
                  pl.BlockSpec((1, H3), lambda i: (0, 0)),
                  pl.BlockSpec((H3, H4), lambda i: (0, 0)),
                  pl.BlockSpec((1, H4), lambda i: (0, 0)),
                  pl.BlockSpec((H4, NC), lambda i: (0, 0)),
                  pl.BlockSpec((1, NC), lambda i: (0, 0))],
        out_specs=pl.BlockSpec((1, NC), lambda i: (0, 0)),
        out_shape=jax.ShapeDtypeStruct((1, NC), jnp.float32),
    )(c_flat, c_flat, deg_in, b2.reshape(1, -1), Wd1, bd1.reshape(1, -1),
      Wd2, bd2.reshape(1, -1), Wo, bo.reshape(1, -1))


def kernel(x, edge_index, W1, b1, W2, b2, Wd1, bd1, Wd2, bd2, Wo, bo):
    N, D = x.shape
    E = edge_index.shape[1]
    DH = D // 2
    B = 1024
    NP = -(-N // 2048) * 2048

    ei_flat = edge_index.reshape(-1)
    deg2 = _make_deg_kernel(E, NP)(ei_flat)
    deg_out = deg2[:NP].reshape(NP, 1)
    deg_in = deg2[NP:].reshape(NP, 1)

    EPT0 = E // _NTEC
    EPTP = -(-EPT0 // (2 * _EK)) * (2 * _EK)
    pad = EPTP - EPT0
    src2 = jnp.pad(edge_index[0].reshape(_NTEC, EPT0), ((0, 0), (0, pad)))
    pad_rows = (N + (jnp.arange(pad, dtype=jnp.int32) % (NP - N)))
    dst2 = jnp.concatenate(
        [edge_index[1].reshape(_NTEC, EPT0),
         jnp.broadcast_to(pad_rows, (_NTEC, pad))], axis=1)
    ei_pad = jnp.concatenate([src2.reshape(-1), dst2.reshape(-1)])
    EP = _NTEC * EPTP

    msg = _make_msg_kernel(NP, EP, DH)
    g = _tc1(x, W1, deg_out, NP, B)
    a_flat = msg(g, ei_pad)
    h = _tc2(a_flat, deg_in, deg_out, b1.reshape(1, -1), W2, NP, B)
    c_flat = msg(h, ei_pad)
    return _tc3(c_flat, deg_in, b2, Wd1, bd1, Wd2, bd2, Wo, bo, N, NP)


# speedup vs baseline: 1.2433x; 1.2433x over previous
import functools

import jax
import jax.numpy as jnp
from jax import lax
from jax.experimental import pallas as pl
from jax.experimental.pallas import tpu as pltpu
from jax.experimental.pallas import tpu_sc as plsc

_NTEC = 16
_EK = 80


def _sc_mesh():
    return plsc.VectorSubcoreMesh(core_axis_name="c", subcore_axis_name="s")


_SC_PARAMS = pltpu.CompilerParams(needs_layout_passes=False)


def _make_deg_kernel(E, NPAD):
    EPT = E // _NTEC
    RPT = NPAD // _NTEC

    @functools.partial(
        pl.kernel,
        out_type=jax.ShapeDtypeStruct((2 * NPAD,), jnp.float32),
        mesh=_sc_mesh(),
        compiler_params=_SC_PARAMS,
        scratch_types=[
            pltpu.VMEM((NPAD,), jnp.float32),
            pltpu.VMEM_SHARED((_NTEC, NPAD), jnp.float32),
            pltpu.VMEM((EPT,), jnp.int32),
            pltpu.VMEM((_NTEC, RPT), jnp.float32),
            pltpu.VMEM((RPT,), jnp.float32),
        ],
    )
    def deg_kernel(ei_hbm, o_hbm, accv, stage_sh, idx_all, pbuf, obuf):
        cid = lax.axis_index("c")
        sid = lax.axis_index("s")
        zeros16 = jnp.zeros((16,), jnp.float32)
        ones16 = jnp.full((16,), 1.0, jnp.float32)

        def _z(i, c):
            accv[pl.ds(i * 16, 16)] = zeros16
            return c
        lax.fori_loop(0, NPAD // 16, _z, 0)

        pltpu.sync_copy(ei_hbm.at[pl.ds(cid * E + sid * EPT, EPT)], idx_all)

        def _e(i, c):
            v = idx_all[pl.ds(i * 16, 16)]
            plsc.addupdate_scatter(accv, [v], ones16)
            return c
        lax.fori_loop(0, EPT // 16, _e, 0)

        pltpu.sync_copy(accv, stage_sh.at[sid])
        plsc.subcore_barrier()
        pltpu.sync_copy(stage_sh.at[:, pl.ds(sid * RPT, RPT)], pbuf)

        def _s(i, c):
            a = pbuf[0, pl.ds(i * 16, 16)]
            for j in range(1, _NTEC):
                a = a + pbuf[j, pl.ds(i * 16, 16)]
            obuf[pl.ds(i * 16, 16)] = a
            return c
        lax.fori_loop(0, RPT // 16, _s, 0)

        pltpu.sync_copy(obuf, o_hbm.at[pl.ds(cid * NPAD + sid * RPT, RPT)])

    return deg_kernel


def _make_msg_kernel(NP, E, DH):
    EPT = E // _NTEC
    NCH = EPT // _EK
    assert (NCH - 2) % 3 == 0, "pipeline assumes 3k+2 chunks per tile"
    RPT = NP // _NTEC
    ZR = 32
    NB = (NCH - 2) // 3

    @functools.partial(
        pl.kernel,
        out_type=jax.ShapeDtypeStruct((2 * NP, DH), jnp.float32),
        mesh=_sc_mesh(),
        compiler_params=_SC_PARAMS,
        scratch_types=[
            pltpu.VMEM_SHARED((NP, DH), jnp.float32),
            pltpu.VMEM((_EK, DH), jnp.float32),
            pltpu.VMEM((_EK, DH), jnp.float32),
            pltpu.VMEM((_EK, DH), jnp.float32),
            pltpu.VMEM((EPT,), jnp.int32),
            pltpu.VMEM((_EK,), jnp.int32),
            pltpu.VMEM((_EK,), jnp.int32),
            pltpu.VMEM((_EK,), jnp.int32),
            pltpu.VMEM((ZR, DH), jnp.float32),
            pltpu.SemaphoreType.DMA,
            pltpu.SemaphoreType.DMA,
            pltpu.SemaphoreType.DMA,
            pltpu.SemaphoreType.DMA,
            pltpu.SemaphoreType.DMA,
            pltpu.SemaphoreType.DMA,
            pltpu.SemaphoreType.DMA,
            pltpu.SemaphoreType.DMA,
            pltpu.SemaphoreType.DMA,
            pltpu.SemaphoreType.DMA,
            pltpu.SemaphoreType.DMA,
        ],
    )
    def msg_kernel(g_hbm, ei_hbm, o_hbm,
                   acc_sh, rowsA, rowsB, rowsC, gidx, dstA, dstB, dstC, zbuf,
                   sI, sZ, sgA, sgB, sgC, ssA, ssB, ssC, sdA, sdB, sdC):
        cid = lax.axis_index("c")
        sid = lax.axis_index("s")
        zeros16 = jnp.zeros((16,), jnp.float32)
        base = sid * EPT

        idx_cp = pltpu.async_copy(ei_hbm.at[pl.ds(base, EPT)], gidx, sI)

        def _z(i, c):
            for m in range(DH // 16):
                zbuf[i, pl.ds(m * 16, 16)] = zeros16
            return c
        lax.fori_loop(0, ZR, _z, 0)
        for k in range(RPT // ZR):
            pltpu.async_copy(zbuf, acc_sh.at[pl.ds(sid * RPT + k * ZR, ZR)], sZ)
        idx_cp.wait()

        cbase = cid * 8

        def _gi(i, c):
            v = gidx[pl.ds(i * 16, 16)]
            gidx[pl.ds(i * 16, 16)] = ((v >> 3) << 4) + (v & 7) + cbase
            return c
        lax.fori_loop(0, EPT // 16, _gi, 0)
        for k in range(RPT // ZR):
            pltpu.make_async_copy(
                zbuf, acc_sh.at[pl.ds(sid * RPT, ZR)], sZ).wait()
        plsc.subcore_barrier()

        def _gather(j, rows, sem):
            pltpu.async_copy(g_hbm.at[gidx.at[pl.ds(j * _EK, _EK)]], rows, sem)

        def _wait_gather(rows, sem):
            pltpu.make_async_copy(
                g_hbm.at[gidx.at[pl.ds(0, _EK)]], rows, sem).wait()

        def _scatter(rows, db, sem):
            pltpu.async_copy(rows, acc_sh.at[db], sem, add=True)

        def _wait_scatter(rows, db, sem):
            pltpu.make_async_copy(rows, acc_sh.at[db], sem).wait()

        def _dstload(j, db, sem):
            pltpu.async_copy(ei_hbm.at[pl.ds(E + base + j * _EK, _EK)], db, sem)

        def _wait_dstload(db, sem):
            pltpu.make_async_copy(
                ei_hbm.at[pl.ds(E, _EK)], db, sem).wait()

        pltpu.sync_copy(ei_hbm.at[pl.ds(E + base, _EK)], dstA)
        pltpu.sync_copy(ei_hbm.at[pl.ds(E + base + _EK, _EK)], dstB)
        _gather(0, rowsA, sgA)
        _gather(1, rowsB, sgB)

        def _triple(i, c):
            j0 = 3 * i

            @pl.when(i > 0)
            def _():
                _wait_scatter(rowsC, dstC, ssC)
            _dstload(j0 + 2, dstC, sdC)
            _gather(j0 + 2, rowsC, sgC)

            _wait_gather(rowsA, sgA)

            @pl.when(i > 0)
            def _():
                _wait_dstload(dstA, sdA)
            _scatter(rowsA, dstA, ssA)

            _wait_gather(rowsB, sgB)

            @pl.when(i > 0)
            def _():
                _wait_dstload(dstB, sdB)
            _scatter(rowsB, dstB, ssB)

            _wait_scatter(rowsA, dstA, ssA)
            _dstload(j0 + 3, dstA, sdA)
            _gather(j0 + 3, rowsA, sgA)

            _wait_scatter(rowsB, dstB, ssB)
            _dstload(j0 + 4, dstB, sdB)
            _gather(j0 + 4, rowsB, sgB)

            _wait_gather(rowsC, sgC)
            _wait_dstload(dstC, sdC)
            _scatter(rowsC, dstC, ssC)
            return c
        lax.fori_loop(0, NB, _triple, 0)

        _wait_gather(rowsA, sgA)
        _wait_dstload(dstA, sdA)
        _scatter(rowsA, dstA, ssA)
        _wait_gather(rowsB, sgB)
        _wait_dstload(dstB, sdB)
        _scatter(rowsB, dstB, ssB)
        _wait_scatter(rowsC, dstC, ssC)
        _wait_scatter(rowsA, dstA, ssA)
        _wait_scatter(rowsB, dstB, ssB)

        plsc.subcore_barrier()
        r0 = sid * RPT
        pltpu.sync_copy(acc_sh.at[pl.ds(r0, RPT)],
                        o_hbm.at[pl.ds(cid * NP + r0, RPT)])

    return msg_kernel


def _half_shuffle(gb, B, DH):
    return gb.reshape(B // 8, 8, 2, DH).swapaxes(1, 2).reshape(2 * B, DH)


def _tc1_body(x_ref, w_ref, deg_ref, o_ref):
    s = lax.rsqrt(jnp.maximum(deg_ref[...], 1.0))
    g = jnp.dot(x_ref[...] * s, w_ref[...],
                preferred_element_type=jnp.float32)
    o_ref[...] = _half_shuffle(g, g.shape[0], g.shape[1] // 2)


def _tc1(x, W1, deg_out, NP, B):
    N, D = x.shape
    nb = NP // B
    return pl.pallas_call(
        _tc1_body,
        grid=(nb,),
        in_specs=[pl.BlockSpec((B, D), lambda i: (i, 0)),
                  pl.BlockSpec((D, D), lambda i: (0, 0)),
                  pl.BlockSpec((B, 1), lambda i: (i, 0))],
        out_specs=pl.BlockSpec((2 * B, D // 2), lambda i: (i, 0)),
        out_shape=jax.ShapeDtypeStruct((2 * NP, D // 2), jnp.float32),
    )(x, W1, deg_out)


def _tc2_body(a0_ref, a1_ref, di_ref, do_ref, b_ref, w_ref, o_ref):
    si = lax.rsqrt(jnp.maximum(di_ref[...], 1.0))
    so = lax.rsqrt(jnp.maximum(do_ref[...], 1.0))
    b = b_ref[...]
    h0 = jax.nn.relu(a0_ref[...] * si + b[:, :128]) * so
    h1 = jax.nn.relu(a1_ref[...] * si + b[:, 128:]) * so
    w = w_ref[...]
    g = (jnp.dot(h0, w[:128, :], preferred_element_type=jnp.float32)
         + jnp.dot(h1, w[128:, :], preferred_element_type=jnp.float32))
    o_ref[...] = _half_shuffle(g, g.shape[0], g.shape[1] // 2)


def _tc2(a_flat, deg_in, deg_out, b1, W2, NP, B):
    D = W2.shape[0]
    nb = NP // B
    return pl.pallas_call(
        _tc2_body,
        grid=(nb,),
        in_specs=[pl.BlockSpec((B, 128), lambda i: (i, 0)),
                  pl.BlockSpec((B, 128), lambda i, _nb=nb: (_nb + i, 0)),
                  pl.BlockSpec((B, 1), lambda i: (i, 0)),
                  pl.BlockSpec((B, 1), lambda i: (i, 0)),
                  pl.BlockSpec((1, D), lambda i: (0, 0)),
                  pl.BlockSpec((D, D), lambda i: (0, 0))],
        out_specs=pl.BlockSpec((2 * B, D // 2), lambda i: (i, 0)),
        out_shape=jax.ShapeDtypeStruct((2 * NP, D // 2), jnp.float32),
    )(a_flat, a_flat, deg_in, deg_out, b1, W2)


def _tc3_body(a0_ref, a1_ref, di_ref, b2_ref, wd1_ref, bd1_ref,
              wd2_ref, bd2_ref, wo_ref, bo_ref, o_ref, *, n_real):
    npad = a0_ref.shape[0]
    si = lax.rsqrt(jnp.maximum(di_ref[...], 1.0))
    mask = (lax.broadcasted_iota(jnp.int32, (npad, 1), 0)
            < n_real).astype(jnp.float32)
    b2 = b2_ref[...]
    h0 = jax.nn.relu(a0_ref[...] * si + b2[:, :128]) * mask
    h1 = jax.nn.relu(a1_ref[...] * si + b2[:, 128:]) * mask
    m0 = jnp.sum(h0, axis=0, keepdims=True) * (1.0 / n_real)
    m1 = jnp.sum(h1, axis=0, keepdims=True) * (1.0 / n_real)
    hg = jnp.concatenate([m0, m1], axis=1)
    z = jax.nn.relu(jnp.dot(hg, wd1_ref[...],
                            preferred_element_type=jnp.float32) + bd1_ref[...])
    z = jax.nn.relu(jnp.dot(z, wd2_ref[...],
                            preferred_element_type=jnp.float32) + bd2_ref[...])
    o_ref[...] = jnp.dot(z, wo_ref[...],
                         preferred_element_type=jnp.float32) + bo_ref[...]


def _tc3(c_flat, deg_in, b2, Wd1, bd1, Wd2, bd2, Wo, bo, N, NP):
    NC = Wo.shape[1]
    H3 = Wd1.shape[1]
    H4 = Wd2.shape[1]
    D = Wd1.shape[0]
    return pl.pallas_call(
        functools.partial(_tc3_body, n_real=N),
        grid=(1,),
        in_specs=[pl.BlockSpec((NP, 128), lambda i: (0, 0)),
                  pl.BlockSpec((NP, 128), lambda i: (1, 0)),
                  pl.BlockSpec((NP, 1), lambda i: (0, 0)),
                  pl.BlockSpec((1, D), lambda i: (0, 0)),
                  pl.BlockSpec((D, H3), lambda i: (0, 0)),
                  pl.BlockSpec((1, H3), lambda i: (0, 0)),
                  pl.BlockSpec((H3, H4), lambda i: (0, 0)),
                  pl.BlockSpec((1, H4), lambda i: (0, 0)),
                  pl.BlockSpec((H4, NC), lambda i: (0, 0)),
                  pl.BlockSpec((1, NC), lambda i: (0, 0))],
        out_specs=pl.BlockSpec((1, NC), lambda i: (0, 0)),
        out_shape=jax.ShapeDtypeStruct((1, NC), jnp.float32),
    )(c_flat, c_flat, deg_in, b2.reshape(1, -1), Wd1, bd1.reshape(1, -1),
      Wd2, bd2.reshape(1, -1), Wo, bo.reshape(1, -1))


def kernel(x, edge_index, W1, b1, W2, b2, Wd1, bd1, Wd2, bd2, Wo, bo):
    N, D = x.shape
    E = edge_index.shape[1]
    DH = D // 2
    B = 1024
    NP = -(-N // 2048) * 2048

    ei_flat = edge_index.reshape(-1)
    deg2 = _make_deg_kernel(E, NP)(ei_flat)
    deg_out = deg2[:NP].reshape(NP, 1)
    deg_in = deg2[NP:].reshape(NP, 1)

    msg = _make_msg_kernel(NP, E, DH)
    g = _tc1(x, W1, deg_out, NP, B)
    a_flat = msg(g, ei_flat)
    h = _tc2(a_flat, deg_in, deg_out, b1.reshape(1, -1), W2, NP, B)
    c_flat = msg(h, ei_flat)
    return _tc3(c_flat, deg_in, b2, Wd1, bd1, Wd2, bd2, Wo, bo, N, NP)


# speedup vs baseline: 1.3671x; 1.0995x over previous
import functools

import jax
import jax.numpy as jnp
from jax import lax
from jax.experimental import pallas as pl
from jax.experimental.pallas import tpu as pltpu
from jax.experimental.pallas import tpu_sc as plsc

_NTEC = 16
_EK = 128


def _sc_mesh():
    return plsc.VectorSubcoreMesh(core_axis_name="c", subcore_axis_name="s")


_SC_PARAMS = pltpu.CompilerParams(needs_layout_passes=False)


def _make_deg_kernel(E, NPAD):
    EPT = E // _NTEC
    RPT = NPAD // _NTEC

    @functools.partial(
        pl.kernel,
        out_type=jax.ShapeDtypeStruct((2 * NPAD,), jnp.float32),
        mesh=_sc_mesh(),
        compiler_params=_SC_PARAMS,
        scratch_types=[
            pltpu.VMEM((NPAD,), jnp.float32),
            pltpu.VMEM_SHARED((_NTEC, NPAD), jnp.float32),
            pltpu.VMEM((EPT,), jnp.int32),
            pltpu.VMEM((_NTEC, RPT), jnp.float32),
            pltpu.VMEM((RPT,), jnp.float32),
        ],
    )
    def deg_kernel(ei_hbm, o_hbm, accv, stage_sh, idx_all, pbuf, obuf):
        cid = lax.axis_index("c")
        sid = lax.axis_index("s")
        zeros16 = jnp.zeros((16,), jnp.float32)
        ones16 = jnp.full((16,), 1.0, jnp.float32)

        def _z(i, c):
            accv[pl.ds(i * 16, 16)] = zeros16
            return c
        lax.fori_loop(0, NPAD // 16, _z, 0)

        pltpu.sync_copy(ei_hbm.at[pl.ds(cid * E + sid * EPT, EPT)], idx_all)

        def _e(i, c):
            v = idx_all[pl.ds(i * 16, 16)]
            plsc.addupdate_scatter(accv, [v], ones16)
            return c
        lax.fori_loop(0, EPT // 16, _e, 0)

        pltpu.sync_copy(accv, stage_sh.at[sid])
        plsc.subcore_barrier()
        pltpu.sync_copy(stage_sh.at[:, pl.ds(sid * RPT, RPT)], pbuf)

        def _s(i, c):
            a = pbuf[0, pl.ds(i * 16, 16)]
            for j in range(1, _NTEC):
                a = a + pbuf[j, pl.ds(i * 16, 16)]
            obuf[pl.ds(i * 16, 16)] = a
            return c
        lax.fori_loop(0, RPT // 16, _s, 0)

        pltpu.sync_copy(obuf, o_hbm.at[pl.ds(cid * NPAD + sid * RPT, RPT)])

    return deg_kernel


def _make_msg_kernel(NP, E, DH):
    EPT = E // _NTEC
    NCH = EPT // _EK
    REM = EPT - NCH * _EK
    assert NCH % 2 == 0 and REM % 8 == 0 and 0 < REM <= _EK
    RPT = NP // _NTEC
    ZR = 16
    NB = (NCH - 2) // 2

    @functools.partial(
        pl.kernel,
        out_type=jax.ShapeDtypeStruct((2 * NP, DH), jnp.float32),
        mesh=_sc_mesh(),
        compiler_params=_SC_PARAMS,
        scratch_types=[
            pltpu.VMEM_SHARED((NP, DH), jnp.float32),
            pltpu.VMEM((_EK, DH), jnp.float32),
            pltpu.VMEM((_EK, DH), jnp.float32),
            pltpu.VMEM((EPT,), jnp.int32),
            pltpu.VMEM((_EK,), jnp.int32),
            pltpu.VMEM((_EK,), jnp.int32),
            pltpu.VMEM((REM,), jnp.int32),
            pltpu.VMEM((ZR, DH), jnp.float32),
            pltpu.SemaphoreType.DMA,
            pltpu.SemaphoreType.DMA,
            pltpu.SemaphoreType.DMA,
            pltpu.SemaphoreType.DMA,
            pltpu.SemaphoreType.DMA,
            pltpu.SemaphoreType.DMA,
            pltpu.SemaphoreType.DMA,
            pltpu.SemaphoreType.DMA,
        ],
    )
    def msg_kernel(g_hbm, ei_hbm, o_hbm,
                   acc_sh, rowsA, rowsB, gidx, dstA, dstB, dstR, zbuf,
                   sI, sZ, sgA, sgB, ssA, ssB, sdA, sdB):
        cid = lax.axis_index("c")
        sid = lax.axis_index("s")
        zeros16 = jnp.zeros((16,), jnp.float32)
        base = sid * EPT

        idx_cp = pltpu.async_copy(ei_hbm.at[pl.ds(base, EPT)], gidx, sI)

        def _z(i, c):
            for m in range(DH // 16):
                zbuf[i, pl.ds(m * 16, 16)] = zeros16
            return c
        lax.fori_loop(0, ZR, _z, 0)
        for k in range(RPT // ZR):
            pltpu.async_copy(zbuf, acc_sh.at[pl.ds(sid * RPT + k * ZR, ZR)], sZ)
        idx_cp.wait()

        cbase = cid * 8

        def _gi(i, c):
            v = gidx[pl.ds(i * 16, 16)]
            gidx[pl.ds(i * 16, 16)] = ((v >> 3) << 4) + (v & 7) + cbase
            return c
        lax.fori_loop(0, EPT // 16, _gi, 0)
        for k in range(RPT // ZR):
            pltpu.make_async_copy(
                zbuf, acc_sh.at[pl.ds(sid * RPT, ZR)], sZ).wait()
        plsc.subcore_barrier()

        def _gather(j, rows, sem):
            pltpu.async_copy(g_hbm.at[gidx.at[pl.ds(j * _EK, _EK)]], rows, sem)

        def _wait_gather(rows, sem):
            pltpu.make_async_copy(
                g_hbm.at[gidx.at[pl.ds(0, _EK)]], rows, sem).wait()

        def _scatter(rows, db, sem):
            pltpu.async_copy(rows, acc_sh.at[db], sem, add=True)

        def _wait_scatter(rows, db, sem):
            pltpu.make_async_copy(rows, acc_sh.at[db], sem).wait()

        def _dstload(j, db, sem):
            pltpu.async_copy(ei_hbm.at[pl.ds(E + base + j * _EK, _EK)], db, sem)

        def _wait_dstload(db, sem):
            pltpu.make_async_copy(
                ei_hbm.at[pl.ds(E, _EK)], db, sem).wait()

        rbase = base + NCH * _EK
        pltpu.sync_copy(ei_hbm.at[pl.ds(E + rbase, REM)], dstR)
        pltpu.async_copy(g_hbm.at[gidx.at[pl.ds(NCH * _EK, REM)]],
                         rowsA.at[pl.ds(0, REM)], sgA).wait()
        pltpu.async_copy(rowsA.at[pl.ds(0, REM)], acc_sh.at[dstR], ssA,
                         add=True)
        pltpu.make_async_copy(rowsA.at[pl.ds(0, REM)], acc_sh.at[dstR],
                              ssA).wait()

        pltpu.sync_copy(ei_hbm.at[pl.ds(E + base, _EK)], dstA)
        _gather(0, rowsA, sgA)

        def _pair(i, c):
            j0 = 2 * i

            @pl.when(i > 0)
            def _():
                _wait_scatter(rowsB, dstB, ssB)
            _dstload(j0 + 1, dstB, sdB)
            _gather(j0 + 1, rowsB, sgB)

            _wait_gather(rowsA, sgA)

            @pl.when(i > 0)
            def _():
                _wait_dstload(dstA, sdA)
            _scatter(rowsA, dstA, ssA)

            _wait_gather(rowsB, sgB)
            _wait_scatter(rowsA, dstA, ssA)
            _dstload(j0 + 2, dstA, sdA)
            _gather(j0 + 2, rowsA, sgA)

            _wait_dstload(dstB, sdB)
            _scatter(rowsB, dstB, ssB)
            return c
        lax.fori_loop(0, NB, _pair, 0)

        _wait_scatter(rowsB, dstB, ssB)
        _dstload(NCH - 1, dstB, sdB)
        _gather(NCH - 1, rowsB, sgB)
        _wait_gather(rowsA, sgA)
        _wait_dstload(dstA, sdA)
        _scatter(rowsA, dstA, ssA)
        _wait_gather(rowsB, sgB)
        _wait_dstload(dstB, sdB)
        _scatter(rowsB, dstB, ssB)
        _wait_scatter(rowsA, dstA, ssA)
        _wait_scatter(rowsB, dstB, ssB)

        plsc.subcore_barrier()
        r0 = sid * RPT
        pltpu.sync_copy(acc_sh.at[pl.ds(r0, RPT)],
                        o_hbm.at[pl.ds(cid * NP + r0, RPT)])

    return msg_kernel


def _half_shuffle(gb, B, DH):
    return gb.reshape(B // 8, 8, 2, DH).swapaxes(1, 2).reshape(2 * B, DH)


def _tc1_body(x_ref, w_ref, deg_ref, o_ref):
    s = lax.rsqrt(jnp.maximum(deg_ref[...], 1.0))
    g = jnp.dot(x_ref[...] * s, w_ref[...],
                preferred_element_type=jnp.float32)
    o_ref[...] = _half_shuffle(g, g.shape[0], g.shape[1] // 2)


def _tc1(x, W1, deg_out, NP, B):
    N, D = x.shape
    nb = NP // B
    return pl.pallas_call(
        _tc1_body,
        grid=(nb,),
        in_specs=[pl.BlockSpec((B, D), lambda i: (i, 0)),
                  pl.BlockSpec((D, D), lambda i: (0, 0)),
                  pl.BlockSpec((B, 1), lambda i: (i, 0))],
        out_specs=pl.BlockSpec((2 * B, D // 2), lambda i: (i, 0)),
        out_shape=jax.ShapeDtypeStruct((2 * NP, D // 2), jnp.float32),
    )(x, W1, deg_out)


def _tc2_body(a0_ref, a1_ref, di_ref, do_ref, b_ref, w_ref, o_ref):
    si = lax.rsqrt(jnp.maximum(di_ref[...], 1.0))
    so = lax.rsqrt(jnp.maximum(do_ref[...], 1.0))
    b = b_ref[...]
    h0 = jax.nn.relu(a0_ref[...] * si + b[:, :128]) * so
    h1 = jax.nn.relu(a1_ref[...] * si + b[:, 128:]) * so
    w = w_ref[...]
    g = (jnp.dot(h0, w[:128, :], preferred_element_type=jnp.float32)
         + jnp.dot(h1, w[128:, :], preferred_element_type=jnp.float32))
    o_ref[...] = _half_shuffle(g, g.shape[0], g.shape[1] // 2)


def _tc2(a_flat, deg_in, deg_out, b1, W2, NP, B):
    D = W2.shape[0]
    nb = NP // B
    return pl.pallas_call(
        _tc2_body,
        grid=(nb,),
        in_specs=[pl.BlockSpec((B, 128), lambda i: (i, 0)),
                  pl.BlockSpec((B, 128), lambda i, _nb=nb: (_nb + i, 0)),
                  pl.BlockSpec((B, 1), lambda i: (i, 0)),
                  pl.BlockSpec((B, 1), lambda i: (i, 0)),
                  pl.BlockSpec((1, D), lambda i: (0, 0)),
                  pl.BlockSpec((D, D), lambda i: (0, 0))],
        out_specs=pl.BlockSpec((2 * B, D // 2), lambda i: (i, 0)),
        out_shape=jax.ShapeDtypeStruct((2 * NP, D // 2), jnp.float32),
    )(a_flat, a_flat, deg_in, deg_out, b1, W2)


def _tc3_body(a0_ref, a1_ref, di_ref, b2_ref, wd1_ref, bd1_ref,
              wd2_ref, bd2_ref, wo_ref, bo_ref, o_ref, *, n_real):
    npad = a0_ref.shape[0]
    si = lax.rsqrt(jnp.maximum(di_ref[...], 1.0))
    mask = (lax.broadcasted_iota(jnp.int32, (npad, 1), 0)
            < n_real).astype(jnp.float32)
    b2 = b2_ref[...]
    h0 = jax.nn.relu(a0_ref[...] * si + b2[:, :128]) * mask
    h1 = jax.nn.relu(a1_ref[...] * si + b2[:, 128:]) * mask
    m0 = jnp.sum(h0, axis=0, keepdims=True) * (1.0 / n_real)
    m1 = jnp.sum(h1, axis=0, keepdims=True) * (1.0 / n_real)
    hg = jnp.concatenate([m0, m1], axis=1)
    z = jax.nn.relu(jnp.dot(hg, wd1_ref[...],
                            preferred_element_type=jnp.float32) + bd1_ref[...])
    z = jax.nn.relu(jnp.dot(z, wd2_ref[...],
                            preferred_element_type=jnp.float32) + bd2_ref[...])
    o_ref[...] = jnp.dot(z, wo_ref[...],
                         preferred_element_type=jnp.float32) + bo_ref[...]


def _tc3(c_flat, deg_in, b2, Wd1, bd1, Wd2, bd2, Wo, bo, N, NP):
    NC = Wo.shape[1]
    H3 = Wd1.shape[1]
    H4 = Wd2.shape[1]
    D = Wd1.shape[0]
    return pl.pallas_call(
        functools.partial(_tc3_body, n_real=N),
        grid=(1,),
        in_specs=[pl.BlockSpec((NP, 128), lambda i: (0, 0)),
                  pl.BlockSpec((NP, 128), lambda i: (1, 0)),
                  pl.BlockSpec((NP, 1), lambda i: (0, 0)),
                  pl.BlockSpec((1, D), lambda i: (0, 0)),
                  pl.BlockSpec((D, H3), lambda i: (0, 0)),
                  pl.BlockSpec((1, H3), lambda i: (0, 0)),
                  pl.BlockSpec((H3, H4), lambda i: (0, 0)),
                  pl.BlockSpec((1, H4), lambda i: (0, 0)),
                  pl.BlockSpec((H4, NC), lambda i: (0, 0)),
                  pl.BlockSpec((1, NC), lambda i: (0, 0))],
        out_specs=pl.BlockSpec((1, NC), lambda i: (0, 0)),
        out_shape=jax.ShapeDtypeStruct((1, NC), jnp.float32),
    )(c_flat, c_flat, deg_in, b2.reshape(1, -1), Wd1, bd1.reshape(1, -1),
      Wd2, bd2.reshape(1, -1), Wo, bo.reshape(1, -1))


def kernel(x, edge_index, W1, b1, W2, b2, Wd1, bd1, Wd2, bd2, Wo, bo):
    N, D = x.shape
    E = edge_index.shape[1]
    DH = D // 2
    B = 1024
    NP = -(-N // 2048) * 2048

    ei_flat = edge_index.reshape(-1)
    deg2 = _make_deg_kernel(E, NP)(ei_flat)
    deg_out = deg2[:NP].reshape(NP, 1)
    deg_in = deg2[NP:].reshape(NP, 1)

    msg = _make_msg_kernel(NP, E, DH)
    g = _tc1(x, W1, deg_out, NP, B)
    a_flat = msg(g, ei_flat)
    h = _tc2(a_flat, deg_in, deg_out, b1.reshape(1, -1), W2, NP, B)
    c_flat = msg(h, ei_flat)
    return _tc3(c_flat, deg_in, b2, Wd1, bd1, Wd2, bd2, Wo, bo, N, NP)
